# Initial kernel scaffold; baseline (speedup 1.0000x reference)
import jax
import jax.numpy as jnp
from jax.experimental import pallas as pl


def kernel(x, edge_index, W0, b0, W1, b1, W2, b2, Wp, bp, Wv, bv):
    raise NotImplementedError("write your pallas kernel here")


# speedup vs baseline: 8.7197x; 8.7197x over previous
import functools

import jax
import jax.numpy as jnp
from jax import lax
from jax.experimental import pallas as pl
from jax.experimental.pallas import tpu as pltpu, tpu_sc as plsc

N_NODES = 10000
N_EDGES = 320000
D_IN = 128
D_H = 128
D_C = 64

NC = 2
NS = 16
NW = NC * NS
K = 128
CHUNKS = 80
E_TILE = CHUNKS * K
E_PAD = NW * E_TILE
N_PAD = 10240
ROWS_PER_TILE = N_PAD // NS

_MESH = plsc.VectorSubcoreMesh(
    core_axis_name="c", subcore_axis_name="s", num_cores=NC, num_subcores=NS)


@functools.partial(
    pl.kernel,
    out_type=jax.ShapeDtypeStruct((NC, N_PAD, D_H), jnp.float32),
    mesh=_MESH,
    scratch_types=[
        pltpu.VMEM((CHUNKS, K), jnp.int32),
        pltpu.VMEM((CHUNKS, K), jnp.int32),
        pltpu.VMEM((K, D_H), jnp.float32),
        pltpu.VMEM_SHARED((N_PAD, D_H), jnp.float32),
    ],
)
def _msgpass(h, src3, dst3, zrows, out, src_v, dst_v, buf, acc):
  c = lax.axis_index("c")
  s = lax.axis_index("s")
  wid = c * NS + s
  pltpu.sync_copy(src3.at[wid], src_v)
  pltpu.sync_copy(dst3.at[wid], dst_v)
  sl = pl.ds(s * ROWS_PER_TILE, ROWS_PER_TILE)
  pltpu.sync_copy(zrows, acc.at[sl])
  plsc.subcore_barrier()

  def step(j, carry):
    pltpu.sync_copy(h.at[src_v.at[j]], buf)
    pltpu.sync_copy(buf, acc.at[dst_v.at[j]], add=True)
    return carry

  lax.fori_loop(0, CHUNKS, step, 0)
  plsc.subcore_barrier()
  pltpu.sync_copy(acc.at[sl], out.at[c, sl])


@functools.partial(
    pl.kernel,
    out_type=(jax.ShapeDtypeStruct((NC, N_PAD), jnp.float32),
              jax.ShapeDtypeStruct((NC, N_PAD), jnp.float32)),
    mesh=_MESH,
    scratch_types=[
        pltpu.VMEM((CHUNKS, K), jnp.int32),
        pltpu.VMEM((CHUNKS, K), jnp.int32),
        pltpu.VMEM((K,), jnp.float32),
        pltpu.VMEM_SHARED((N_PAD,), jnp.float32),
        pltpu.VMEM_SHARED((N_PAD,), jnp.float32),
    ],
)
def _degrees(src3, dst3, ones_h, zrow, out_o, out_i,
             src_v, dst_v, ones_v, acc_o, acc_i):
  c = lax.axis_index("c")
  s = lax.axis_index("s")
  wid = c * NS + s
  pltpu.sync_copy(src3.at[wid], src_v)
  pltpu.sync_copy(dst3.at[wid], dst_v)
  pltpu.sync_copy(ones_h, ones_v)
  sl = pl.ds(s * ROWS_PER_TILE, ROWS_PER_TILE)
  pltpu.sync_copy(zrow, acc_o.at[sl])
  pltpu.sync_copy(zrow, acc_i.at[sl])
  plsc.subcore_barrier()

  def step(j, carry):
    pltpu.sync_copy(ones_v, acc_o.at[src_v.at[j]], add=True)
    pltpu.sync_copy(ones_v, acc_i.at[dst_v.at[j]], add=True)
    return carry

  lax.fori_loop(0, CHUNKS, step, 0)
  plsc.subcore_barrier()
  pltpu.sync_copy(acc_o.at[sl], out_o.at[c, sl])
  pltpu.sync_copy(acc_i.at[sl], out_i.at[c, sl])


def _norm_from(dparts):
  deg = dparts[0] + dparts[1]
  return lax.rsqrt(jnp.maximum(deg, 1.0))[:, None]


def _tc_prep_body(x_ref, dgo_ref, o_ref):
  o_ref[...] = x_ref[...] * _norm_from(dgo_ref[...])


_tc_prep = pl.pallas_call(
    _tc_prep_body,
    out_shape=jax.ShapeDtypeStruct((N_PAD, D_IN), jnp.float32))


def _tc_layer_body(p_ref, dgo_ref, dgi_ref, w_ref, b_ref, o_ref):
  p = p_ref[...]
  agg = (p[0] + p[1]) * _norm_from(dgi_ref[...])
  h = jnp.dot(agg, w_ref[...], preferred_element_type=jnp.float32) + b_ref[...]
  o_ref[...] = jnp.maximum(h, 0.0) * _norm_from(dgo_ref[...])


_tc_layer = pl.pallas_call(
    _tc_layer_body,
    out_shape=jax.ShapeDtypeStruct((N_PAD, D_H), jnp.float32))


def _tc_final_body(p_ref, dgi_ref, w2_ref, b2_ref, wp_ref, bp_ref, wv_ref,
                   bv_ref, pi_ref, v_ref):
  p = p_ref[...]
  agg = (p[0] + p[1]) * _norm_from(dgi_ref[...])
  h3 = (jnp.dot(agg, w2_ref[...], preferred_element_type=jnp.float32)
        + b2_ref[...])
  pi_ref[...] = (
      jnp.dot(h3, wp_ref[...], preferred_element_type=jnp.float32)
      + bp_ref[...])
  m = jnp.sum(h3[:N_NODES, :], axis=0, keepdims=True) * (1.0 / N_NODES)
  v_ref[...] = (
      jnp.dot(m, wv_ref[...], preferred_element_type=jnp.float32)
      + bv_ref[...])


_tc_final = pl.pallas_call(
    _tc_final_body,
    out_shape=(jax.ShapeDtypeStruct((N_PAD, 1), jnp.float32),
               jax.ShapeDtypeStruct((1, 1), jnp.float32)))


def kernel(x, edge_index, W0, b0, W1, b1, W2, b2, Wp, bp, Wv, bv):
  src = edge_index[0].astype(jnp.int32)
  dst = edge_index[1].astype(jnp.int32)
  n_extra = E_PAD - N_EDGES
  pad_idx = N_NODES + (jnp.arange(n_extra, dtype=jnp.int32)
                       % (N_PAD - N_NODES))
  src3 = jnp.concatenate([src, pad_idx]).reshape(NW, CHUNKS, K)
  dst3 = jnp.concatenate([dst, pad_idx]).reshape(NW, CHUNKS, K)
  x_pad = jnp.zeros((N_PAD, D_IN), jnp.float32).at[:N_NODES].set(x)

  ones_k = jnp.ones((K,), jnp.float32)
  z_row = jnp.zeros((ROWS_PER_TILE,), jnp.float32)
  z_h = jnp.zeros((ROWS_PER_TILE, D_H), jnp.float32)

  dego, degi = _degrees(src3, dst3, ones_k, z_row)

  h0s = _tc_prep(x_pad, dego)
  p1 = _msgpass(h0s, src3, dst3, z_h)
  h1s = _tc_layer(p1, dego, degi, W0, b0.reshape(1, D_H))
  p2 = _msgpass(h1s, src3, dst3, z_h)
  h2s = _tc_layer(p2, dego, degi, W1, b1.reshape(1, D_H))
  p3 = _msgpass(h2s, src3, dst3, z_h)
  PI, V = _tc_final(p3, degi, W2, b2.reshape(1, D_C), Wp, bp.reshape(1, 1),
                    Wv, bv.reshape(1, 1))
  return (PI[:N_NODES], V)


# speedup vs baseline: 10.9635x; 1.2573x over previous
import functools

import jax
import jax.numpy as jnp
from jax import lax
from jax.experimental import pallas as pl
from jax.experimental.pallas import tpu as pltpu, tpu_sc as plsc

N_NODES = 10000
N_EDGES = 320000
D_IN = 128
D_H = 128
D_C = 64

NC = 2
NS = 16
NW = NC * NS
K = 128
CHUNKS = 80
E_TILE = CHUNKS * K
E_PAD = NW * E_TILE
N_PAD = 10240
ROWS_PER_TILE = N_PAD // NS

_MESH = plsc.VectorSubcoreMesh(
    core_axis_name="c", subcore_axis_name="s", num_cores=NC, num_subcores=NS)


@functools.partial(
    pl.kernel,
    out_type=jax.ShapeDtypeStruct((NC, N_PAD, D_H), jnp.float32),
    mesh=_MESH,
    scratch_types=[
        pltpu.VMEM((CHUNKS // 2, K), jnp.int32),
        pltpu.VMEM((CHUNKS // 2, K), jnp.int32),
        pltpu.VMEM((K, D_H), jnp.float32),
        pltpu.VMEM((K, D_H), jnp.float32),
        pltpu.VMEM_SHARED((N_PAD, D_H), jnp.float32),
        pltpu.SemaphoreType.DMA,
        pltpu.SemaphoreType.DMA,
    ],
)
def _msgpass(h, src3, dst3, zrows, out, src_v, dst_v, buf0, buf1, acc,
             sem0, sem1):
  c = lax.axis_index("c")
  s = lax.axis_index("s")
  wid = c * NS + s
  sl = pl.ds(s * ROWS_PER_TILE, ROWS_PER_TILE)
  pltpu.sync_copy(zrows, acc.at[sl])
  plsc.subcore_barrier()

  half = CHUNKS // 2

  def step(i, carry):
    j = 2 * i
    pltpu.make_async_copy(h.at[src_v.at[j]], buf0, sem0).wait()
    pltpu.async_copy(h.at[src_v.at[j + 1]], buf1, sem1)
    pltpu.sync_copy(buf0, acc.at[dst_v.at[j]], add=True)
    pltpu.make_async_copy(h.at[src_v.at[j + 1]], buf1, sem1).wait()
    jn = jnp.minimum(j + 2, half - 1)
    pltpu.async_copy(h.at[src_v.at[jn]], buf0, sem0)
    pltpu.sync_copy(buf1, acc.at[dst_v.at[j + 1]], add=True)
    return carry

  for hbase in (0, half):
    pltpu.sync_copy(src3.at[wid, pl.ds(hbase, half)], src_v)
    pltpu.sync_copy(dst3.at[wid, pl.ds(hbase, half)], dst_v)
    pltpu.async_copy(h.at[src_v.at[0]], buf0, sem0)
    lax.fori_loop(0, half // 2, step, 0)
    pltpu.make_async_copy(h.at[src_v.at[half - 1]], buf0, sem0).wait()

  plsc.subcore_barrier()
  pltpu.sync_copy(acc.at[sl], out.at[c, sl])


@functools.partial(
    pl.kernel,
    out_type=(jax.ShapeDtypeStruct((NC, N_PAD), jnp.float32),
              jax.ShapeDtypeStruct((NC, N_PAD), jnp.float32)),
    mesh=_MESH,
    scratch_types=[
        pltpu.VMEM((CHUNKS, K), jnp.int32),
        pltpu.VMEM((CHUNKS, K), jnp.int32),
        pltpu.VMEM((K,), jnp.float32),
        pltpu.VMEM_SHARED((N_PAD,), jnp.float32),
        pltpu.VMEM_SHARED((N_PAD,), jnp.float32),
    ],
)
def _degrees(src3, dst3, ones_h, zrow, out_o, out_i,
             src_v, dst_v, ones_v, acc_o, acc_i):
  c = lax.axis_index("c")
  s = lax.axis_index("s")
  wid = c * NS + s
  pltpu.sync_copy(src3.at[wid], src_v)
  pltpu.sync_copy(dst3.at[wid], dst_v)
  pltpu.sync_copy(ones_h, ones_v)
  sl = pl.ds(s * ROWS_PER_TILE, ROWS_PER_TILE)
  pltpu.sync_copy(zrow, acc_o.at[sl])
  pltpu.sync_copy(zrow, acc_i.at[sl])
  plsc.subcore_barrier()

  def step(j, carry):
    pltpu.sync_copy(ones_v, acc_o.at[src_v.at[j]], add=True)
    pltpu.sync_copy(ones_v, acc_i.at[dst_v.at[j]], add=True)
    return carry

  lax.fori_loop(0, CHUNKS, step, 0)
  plsc.subcore_barrier()
  pltpu.sync_copy(acc_o.at[sl], out_o.at[c, sl])
  pltpu.sync_copy(acc_i.at[sl], out_i.at[c, sl])


def _norm_from(dparts):
  deg = dparts[0] + dparts[1]
  return lax.rsqrt(jnp.maximum(deg, 1.0))[:, None]


def _tc_prep_body(x_ref, dgo_ref, o_ref):
  o_ref[...] = x_ref[...] * _norm_from(dgo_ref[...])


_tc_prep = pl.pallas_call(
    _tc_prep_body,
    out_shape=jax.ShapeDtypeStruct((N_PAD, D_IN), jnp.float32))


def _tc_layer_body(p_ref, dgo_ref, dgi_ref, w_ref, b_ref, o_ref):
  p = p_ref[...]
  agg = (p[0] + p[1]) * _norm_from(dgi_ref[...])
  h = jnp.dot(agg, w_ref[...], preferred_element_type=jnp.float32) + b_ref[...]
  o_ref[...] = jnp.maximum(h, 0.0) * _norm_from(dgo_ref[...])


_tc_layer = pl.pallas_call(
    _tc_layer_body,
    out_shape=jax.ShapeDtypeStruct((N_PAD, D_H), jnp.float32))


def _tc_final_body(p_ref, dgi_ref, w2_ref, b2_ref, wp_ref, bp_ref, wv_ref,
                   bv_ref, pi_ref, v_ref):
  p = p_ref[...]
  agg = (p[0] + p[1]) * _norm_from(dgi_ref[...])
  h3 = (jnp.dot(agg, w2_ref[...], preferred_element_type=jnp.float32)
        + b2_ref[...])
  pi_ref[...] = (
      jnp.dot(h3, wp_ref[...], preferred_element_type=jnp.float32)
      + bp_ref[...])
  m = jnp.sum(h3[:N_NODES, :], axis=0, keepdims=True) * (1.0 / N_NODES)
  v_ref[...] = (
      jnp.dot(m, wv_ref[...], preferred_element_type=jnp.float32)
      + bv_ref[...])


_tc_final = pl.pallas_call(
    _tc_final_body,
    out_shape=(jax.ShapeDtypeStruct((N_PAD, 1), jnp.float32),
               jax.ShapeDtypeStruct((1, 1), jnp.float32)))


def kernel(x, edge_index, W0, b0, W1, b1, W2, b2, Wp, bp, Wv, bv):
  src = edge_index[0].astype(jnp.int32)
  dst = edge_index[1].astype(jnp.int32)
  n_extra = E_PAD - N_EDGES
  pad_idx = N_NODES + (jnp.arange(n_extra, dtype=jnp.int32)
                       % (N_PAD - N_NODES))
  src3 = jnp.concatenate([src, pad_idx]).reshape(NW, CHUNKS, K)
  dst3 = jnp.concatenate([dst, pad_idx]).reshape(NW, CHUNKS, K)
  x_pad = jnp.zeros((N_PAD, D_IN), jnp.float32).at[:N_NODES].set(x)

  ones_k = jnp.ones((K,), jnp.float32)
  z_row = jnp.zeros((ROWS_PER_TILE,), jnp.float32)
  z_h = jnp.zeros((ROWS_PER_TILE, D_H), jnp.float32)

  dego, degi = _degrees(src3, dst3, ones_k, z_row)

  h0s = _tc_prep(x_pad, dego)
  p1 = _msgpass(h0s, src3, dst3, z_h)
  h1s = _tc_layer(p1, dego, degi, W0, b0.reshape(1, D_H))
  p2 = _msgpass(h1s, src3, dst3, z_h)
  h2s = _tc_layer(p2, dego, degi, W1, b1.reshape(1, D_H))
  p3 = _msgpass(h2s, src3, dst3, z_h)
  PI, V = _tc_final(p3, degi, W2, b2.reshape(1, D_C), Wp, bp.reshape(1, 1),
                    Wv, bv.reshape(1, 1))
  return (PI[:N_NODES], V)


# speedup vs baseline: 11.6487x; 1.0625x over previous
import functools

import jax
import jax.numpy as jnp
from jax import lax
from jax.experimental import pallas as pl
from jax.experimental.pallas import tpu as pltpu, tpu_sc as plsc

N_NODES = 10000
N_EDGES = 320000
D_IN = 128
D_H = 128
D_C = 64

NC = 2
NS = 16
NW = NC * NS
K = 128
CHUNKS = 80
E_TILE = CHUNKS * K
E_PAD = NW * E_TILE
N_PAD = 10240
ROWS_PER_TILE = N_PAD // NS

_MESH = plsc.VectorSubcoreMesh(
    core_axis_name="c", subcore_axis_name="s", num_cores=NC, num_subcores=NS)


@functools.partial(
    pl.kernel,
    out_type=jax.ShapeDtypeStruct((NC, N_PAD, D_H), jnp.float32),
    mesh=_MESH,
    scratch_types=[
        pltpu.VMEM((CHUNKS // 2, K), jnp.int32),
        pltpu.VMEM((CHUNKS // 2, K), jnp.int32),
        pltpu.VMEM((K, D_H), jnp.float32),
        pltpu.VMEM((K, D_H), jnp.float32),
        pltpu.VMEM_SHARED((N_PAD, D_H), jnp.float32),
        pltpu.SemaphoreType.DMA,
        pltpu.SemaphoreType.DMA,
        pltpu.SemaphoreType.DMA,
        pltpu.SemaphoreType.DMA,
    ],
)
def _msgpass(h, src3, dst3, zrows, out, src_v, dst_v, buf0, buf1, acc,
             sem0, sem1, ssem0, ssem1):
  c = lax.axis_index("c")
  s = lax.axis_index("s")
  wid = c * NS + s
  sl = pl.ds(s * ROWS_PER_TILE, ROWS_PER_TILE)
  pltpu.sync_copy(zrows, acc.at[sl])
  plsc.subcore_barrier()

  half = CHUNKS // 2

  def step(i, carry):
    j = 2 * i
    pltpu.make_async_copy(h.at[src_v.at[j]], buf0, sem0).wait()
    pltpu.async_copy(h.at[src_v.at[j + 1]], buf1, sem1)
    pltpu.sync_copy(buf0, acc.at[dst_v.at[j]], add=True)
    pltpu.make_async_copy(h.at[src_v.at[j + 1]], buf1, sem1).wait()
    jn = jnp.minimum(j + 2, half - 1)
    pltpu.async_copy(h.at[src_v.at[jn]], buf0, sem0)
    pltpu.sync_copy(buf1, acc.at[dst_v.at[j + 1]], add=True)
    return carry

  for hbase in (0, half):
    pltpu.sync_copy(src3.at[wid, pl.ds(hbase, half)], src_v)
    pltpu.sync_copy(dst3.at[wid, pl.ds(hbase, half)], dst_v)
    pltpu.async_copy(h.at[src_v.at[0]], buf0, sem0)
    lax.fori_loop(0, half // 2, step, 0)
    pltpu.make_async_copy(h.at[src_v.at[half - 1]], buf0, sem0).wait()

  plsc.subcore_barrier()
  pltpu.sync_copy(acc.at[sl], out.at[c, sl])


@functools.partial(
    pl.kernel,
    out_type=jax.ShapeDtypeStruct((NC, N_PAD, D_C), jnp.float32),
    mesh=_MESH,
    scratch_types=[
        pltpu.VMEM((CHUNKS, K), jnp.int32),
        pltpu.VMEM((CHUNKS, K), jnp.int32),
        pltpu.VMEM((K, D_C), jnp.float32),
        pltpu.VMEM((K, D_C), jnp.float32),
        pltpu.VMEM_SHARED((N_PAD, D_C), jnp.float32),
        pltpu.SemaphoreType.DMA,
        pltpu.SemaphoreType.DMA,
    ],
    compiler_params=pltpu.CompilerParams(use_tc_tiling_on_sc=False),
)
def _msgpass64(h, src3, dst3, zrows, out, src_v, dst_v, buf0, buf1, acc,
               sem0, sem1):
  c = lax.axis_index("c")
  s = lax.axis_index("s")
  wid = c * NS + s
  sl = pl.ds(s * ROWS_PER_TILE, ROWS_PER_TILE)
  pltpu.sync_copy(zrows, acc.at[sl])
  pltpu.sync_copy(src3.at[wid], src_v)
  pltpu.sync_copy(dst3.at[wid], dst_v)
  plsc.subcore_barrier()

  def step(i, carry):
    j = 2 * i
    pltpu.make_async_copy(h.at[src_v.at[j]], buf0, sem0).wait()
    pltpu.async_copy(h.at[src_v.at[j + 1]], buf1, sem1)
    pltpu.sync_copy(buf0, acc.at[dst_v.at[j]], add=True)
    pltpu.make_async_copy(h.at[src_v.at[j + 1]], buf1, sem1).wait()
    jn = jnp.minimum(j + 2, CHUNKS - 1)
    pltpu.async_copy(h.at[src_v.at[jn]], buf0, sem0)
    pltpu.sync_copy(buf1, acc.at[dst_v.at[j + 1]], add=True)
    return carry

  pltpu.async_copy(h.at[src_v.at[0]], buf0, sem0)
  lax.fori_loop(0, CHUNKS // 2, step, 0)
  pltpu.make_async_copy(h.at[src_v.at[CHUNKS - 1]], buf0, sem0).wait()

  plsc.subcore_barrier()
  pltpu.sync_copy(acc.at[sl], out.at[c, sl])


@functools.partial(
    pl.kernel,
    out_type=(jax.ShapeDtypeStruct((NC, N_PAD), jnp.float32),
              jax.ShapeDtypeStruct((NC, N_PAD), jnp.float32)),
    mesh=_MESH,
    scratch_types=[
        pltpu.VMEM((CHUNKS, K), jnp.int32),
        pltpu.VMEM((CHUNKS, K), jnp.int32),
        pltpu.VMEM((K,), jnp.float32),
        pltpu.VMEM_SHARED((N_PAD,), jnp.float32),
        pltpu.VMEM_SHARED((N_PAD,), jnp.float32),
    ],
)
def _degrees(src3, dst3, ones_h, zrow, out_o, out_i,
             src_v, dst_v, ones_v, acc_o, acc_i):
  c = lax.axis_index("c")
  s = lax.axis_index("s")
  wid = c * NS + s
  pltpu.sync_copy(src3.at[wid], src_v)
  pltpu.sync_copy(dst3.at[wid], dst_v)
  pltpu.sync_copy(ones_h, ones_v)
  sl = pl.ds(s * ROWS_PER_TILE, ROWS_PER_TILE)
  pltpu.sync_copy(zrow, acc_o.at[sl])
  pltpu.sync_copy(zrow, acc_i.at[sl])
  plsc.subcore_barrier()

  def step(j, carry):
    pltpu.sync_copy(ones_v, acc_o.at[src_v.at[j]], add=True)
    pltpu.sync_copy(ones_v, acc_i.at[dst_v.at[j]], add=True)
    return carry

  lax.fori_loop(0, CHUNKS, step, 0)
  plsc.subcore_barrier()
  pltpu.sync_copy(acc_o.at[sl], out_o.at[c, sl])
  pltpu.sync_copy(acc_i.at[sl], out_i.at[c, sl])


def _norm_from(dparts):
  deg = dparts[0] + dparts[1]
  return lax.rsqrt(jnp.maximum(deg, 1.0))[:, None]


def _tc_prep_body(x_ref, dgo_ref, o_ref):
  o_ref[...] = x_ref[...] * _norm_from(dgo_ref[...])


_tc_prep = pl.pallas_call(
    _tc_prep_body,
    out_shape=jax.ShapeDtypeStruct((N_PAD, D_IN), jnp.float32))


def _tc_layer_body(p_ref, dgo_ref, dgi_ref, w_ref, b_ref, o_ref):
  p = p_ref[...]
  agg = (p[0] + p[1]) * _norm_from(dgi_ref[...])
  h = jnp.dot(agg, w_ref[...], preferred_element_type=jnp.float32) + b_ref[...]
  o_ref[...] = jnp.maximum(h, 0.0) * _norm_from(dgo_ref[...])


_tc_layer = pl.pallas_call(
    _tc_layer_body,
    out_shape=jax.ShapeDtypeStruct((N_PAD, D_H), jnp.float32))


def _tc_layer2_body(p_ref, dgo_ref, dgi_ref, w_ref, b_ref, w2_ref, o_ref):
  p = p_ref[...]
  agg = (p[0] + p[1]) * _norm_from(dgi_ref[...])
  h = jnp.dot(agg, w_ref[...], preferred_element_type=jnp.float32) + b_ref[...]
  h = jnp.maximum(h, 0.0) * _norm_from(dgo_ref[...])
  o_ref[...] = jnp.dot(h, w2_ref[...], preferred_element_type=jnp.float32)


_tc_layer2 = pl.pallas_call(
    _tc_layer2_body,
    out_shape=jax.ShapeDtypeStruct((N_PAD, D_C), jnp.float32))


def _tc_final_body(p_ref, dgi_ref, b2_ref, wp_ref, bp_ref, wv_ref,
                   bv_ref, pi_ref, v_ref):
  p = p_ref[...]
  h3 = (p[0] + p[1]) * _norm_from(dgi_ref[...]) + b2_ref[...]
  pi_ref[...] = (
      jnp.dot(h3, wp_ref[...], preferred_element_type=jnp.float32)
      + bp_ref[...])
  m = jnp.sum(h3[:N_NODES, :], axis=0, keepdims=True) * (1.0 / N_NODES)
  v_ref[...] = (
      jnp.dot(m, wv_ref[...], preferred_element_type=jnp.float32)
      + bv_ref[...])


_tc_final = pl.pallas_call(
    _tc_final_body,
    out_shape=(jax.ShapeDtypeStruct((N_PAD, 1), jnp.float32),
               jax.ShapeDtypeStruct((1, 1), jnp.float32)))


def kernel(x, edge_index, W0, b0, W1, b1, W2, b2, Wp, bp, Wv, bv):
  src = edge_index[0].astype(jnp.int32)
  dst = edge_index[1].astype(jnp.int32)
  n_extra = E_PAD - N_EDGES
  pad_idx = N_NODES + (jnp.arange(n_extra, dtype=jnp.int32)
                       % (N_PAD - N_NODES))
  src3 = jnp.concatenate([src, pad_idx]).reshape(NW, CHUNKS, K)
  dst3 = jnp.concatenate([dst, pad_idx]).reshape(NW, CHUNKS, K)
  x_pad = jnp.zeros((N_PAD, D_IN), jnp.float32).at[:N_NODES].set(x)

  ones_k = jnp.ones((K,), jnp.float32)
  z_row = jnp.zeros((ROWS_PER_TILE,), jnp.float32)
  z_h = jnp.zeros((ROWS_PER_TILE, D_H), jnp.float32)
  z_c = jnp.zeros((ROWS_PER_TILE, D_C), jnp.float32)

  dego, degi = _degrees(src3, dst3, ones_k, z_row)

  h0s = _tc_prep(x_pad, dego)
  p1 = _msgpass(h0s, src3, dst3, z_h)
  h1s = _tc_layer(p1, dego, degi, W0, b0.reshape(1, D_H))
  p2 = _msgpass(h1s, src3, dst3, z_h)
  h2s = _tc_layer2(p2, dego, degi, W1, b1.reshape(1, D_H), W2)
  p3 = _msgpass64(h2s, src3, dst3, z_c)
  PI, V = _tc_final(p3, degi, b2.reshape(1, D_C), Wp, bp.reshape(1, 1),
                    Wv, bv.reshape(1, 1))
  return (PI[:N_NODES], V)
